# baseline (speedup 1.0000x reference)
import jax
import jax.numpy as jnp
from jax import lax
from jax.experimental import pallas as pl
from jax.experimental.pallas import tpu as pltpu
from jax.experimental.pallas import tpu_sc as plsc

N = 10000
R = 10240
TRASH = N
E = 320000
NSUB = 16
NTILES = 32
CH = 128
NCHUNK = 80
EPAD = NTILES * NCHUNK * CH
IPAD = NCHUNK + 8
RPT = R // NSUB
NGRID = 8
BLK = R // NGRID


def _make_agg(d):
  if d == 128:
    CHL, NB = 64, 2
  else:
    CHL, NB = 128, 4
  NCH = (EPAD // NTILES) // CHL
  mesh = plsc.VectorSubcoreMesh(core_axis_name="c", subcore_axis_name="s")

  def body(table, srcs, dsts, zeros, out, src_v, dst_v, bufs, acc, gsem,
           ssem):
    c = lax.axis_index("c")
    s = lax.axis_index("s")
    w = c * NSUB + s
    pltpu.sync_copy(srcs.at[w], src_v)
    pltpu.sync_copy(dsts.at[w], dst_v)
    pltpu.sync_copy(zeros, acc.at[pl.ds(s * RPT, RPT)])
    plsc.subcore_barrier()

    for b in range(1, NB):
      pltpu.sync_copy(zeros.at[pl.ds(0, CHL)], bufs.at[b])
    if NB == 2:
      pltpu.async_copy(table.at[src_v.at[0]], bufs.at[0], gsem.at[0])
      pltpu.async_copy(bufs.at[1], acc.at[dst_v.at[0]], ssem.at[1], add=True)
    else:
      pltpu.async_copy(table.at[src_v.at[0]], bufs.at[0], gsem.at[0])
      pltpu.async_copy(table.at[src_v.at[1]], bufs.at[1], gsem.at[1])
      pltpu.async_copy(bufs.at[2], acc.at[dst_v.at[0]], ssem.at[2], add=True)
      pltpu.async_copy(bufs.at[3], acc.at[dst_v.at[1]], ssem.at[3], add=True)

    if NB == 2:
      def group(g, carry):
        j0 = g * 2
        for u in range(2):
          j = j0 + u
          b1 = 1 - u
          pltpu.make_async_copy(table.at[src_v.at[j]], bufs.at[u],
                                gsem.at[u]).wait()
          pltpu.async_copy(bufs.at[u], acc.at[dst_v.at[j]], ssem.at[u],
                           add=True)
          pltpu.make_async_copy(bufs.at[b1], acc.at[dst_v.at[0]],
                                ssem.at[b1]).wait()
          pltpu.async_copy(table.at[src_v.at[j + 1]], bufs.at[b1],
                           gsem.at[b1])
        return carry

      lax.fori_loop(0, NCH // 2, group, 0)
      pltpu.make_async_copy(table.at[src_v.at[0]], bufs.at[0],
                            gsem.at[0]).wait()
      pltpu.make_async_copy(bufs.at[1], acc.at[dst_v.at[0]],
                            ssem.at[1]).wait()
    else:
      def group(g, carry):
        j0 = g * 4
        for u in range(4):
          j = j0 + u
          b2 = (u + 2) % 4
          pltpu.make_async_copy(table.at[src_v.at[j]], bufs.at[u],
                                gsem.at[u]).wait()
          pltpu.async_copy(bufs.at[u], acc.at[dst_v.at[j]], ssem.at[u],
                           add=True)
          pltpu.make_async_copy(bufs.at[b2], acc.at[dst_v.at[0]],
                                ssem.at[b2]).wait()
          pltpu.async_copy(table.at[src_v.at[j + 2]], bufs.at[b2],
                           gsem.at[b2])
        return carry

      lax.fori_loop(0, NCH // 4, group, 0)
      for b in (0, 1):
        pltpu.make_async_copy(table.at[src_v.at[0]], bufs.at[b],
                              gsem.at[b]).wait()
      for b in (2, 3):
        pltpu.make_async_copy(bufs.at[b], acc.at[dst_v.at[0]],
                              ssem.at[b]).wait()

    plsc.subcore_barrier()
    pltpu.sync_copy(acc.at[pl.ds(s * RPT, RPT)],
                    out.at[c, pl.ds(s * RPT, RPT)])

  return pl.kernel(
      body,
      out_type=jax.ShapeDtypeStruct((2, R, d), jnp.float32),
      mesh=mesh,
      compiler_params=pltpu.CompilerParams(use_tc_tiling_on_sc=False),
      scratch_types=[
          pltpu.VMEM((NCH + NB, CHL), jnp.int32),
          pltpu.VMEM((NCH, CHL), jnp.int32),
          pltpu.VMEM((NB, CHL, d), jnp.float32),
          pltpu.VMEM_SHARED((R, d), jnp.float32),
          pltpu.SemaphoreType.DMA((NB,)),
          pltpu.SemaphoreType.DMA((NB,)),
      ],
  )


_agg = {d: _make_agg(d) for d in (16, 32, 64, 128)}


def _node(d):
  return pl.BlockSpec((BLK, d), lambda i: (i, 0))


def _part(d):
  return pl.BlockSpec((2, BLK, d), lambda i: (0, i, 0))


def _whole(shape):
  nd = len(shape)
  return pl.BlockSpec(shape, lambda i: (0,) * nd)


def _tc(body, in_specs, out_specs, out_shape):
  return pl.pallas_call(body, grid=(NGRID,), in_specs=in_specs,
                        out_specs=out_specs, out_shape=out_shape)


def _relu(v):
  return jnp.maximum(v, 0.0)


def _dot(a, b):
  return jax.lax.dot(a, b, preferred_element_type=jnp.float32)


def _degv(dg):
  return jnp.maximum(dg[0][:, 0:1] + dg[1][:, 0:1], 1.0)


def _c0(xr, p0, dg, wl, bl, wr, h1a, h1b, dgv):
  deg = _degv(dg)
  dgv[...] = jnp.broadcast_to(deg, dgv.shape)
  a = (p0[0] + p0[1]) / deg
  h = _relu(_dot(a, wl[...]) + bl[...] + _dot(xr[...], wr[...]))
  h1a[...] = h[:, :128]
  h1b[...] = h[:, 128:]


def _c256(ha, hb, pa, pb, dgv, wl, bl, wr, *outs):
  deg = dgv[:, 0:1]
  a = jnp.concatenate([(pa[0] + pa[1]) / deg, (pb[0] + pb[1]) / deg], axis=1)
  hfull = jnp.concatenate([ha[...], hb[...]], axis=1)
  h = _relu(_dot(a, wl[...]) + bl[...] + _dot(hfull, wr[...]))
  if len(outs) == 2:
    outs[0][...] = h[:, :128]
    outs[1][...] = h[:, 128:]
  else:
    outs[0][...] = h


def _cn(hp, pp, dgv, wl, bl, wr, hn):
  a = (pp[0] + pp[1]) / dgv[:, 0:1]
  hn[...] = _relu(_dot(a, wl[...]) + bl[...] + _dot(hp[...], wr[...]))


def _c7(h7, p7, dgv, wl, bl, wr, wreg8, breg, y8):
  a = (p7[0] + p7[1]) / dgv[:, 0:1]
  h = _relu(_dot(a, wl[...]) + bl[...] + _dot(h7[...], wr[...]))
  y8[...] = _dot(h, wreg8[...]) + breg[...]


def kernel(x, edge_index,
           Wl0, bl0, Wr0, Wl1, bl1, Wr1, Wl2, bl2, Wr2, Wl3, bl3, Wr3,
           Wl4, bl4, Wr4, Wl5, bl5, Wr5, Wl6, bl6, Wr6, Wl7, bl7, Wr7,
           Wreg, breg):
  f32 = jnp.float32
  src = edge_index[0]
  dst = edge_index[1]
  pad = EPAD - E
  srcp = jnp.concatenate([src, jnp.zeros((pad,), jnp.int32)])
  dstp = jnp.concatenate([dst, jnp.full((pad,), TRASH, jnp.int32)])
  idx = {}
  for chl, nb in ((64, 2), (128, 4)):
    nch = (EPAD // NTILES) // chl
    ss = jnp.concatenate(
        [srcp.reshape(NTILES, nch, chl),
         jnp.zeros((NTILES, nb, chl), jnp.int32)], axis=1)
    idx[chl] = (ss, dstp.reshape(NTILES, nch, chl))

  xp = jnp.zeros((R, 128), f32).at[:N].set(x)
  ones16 = jnp.ones((R, 16), f32)
  zer = {d: jnp.zeros((RPT, d), f32) for d in (16, 32, 64, 128)}
  b = {i: v.reshape(1, -1) for i, v in
       enumerate([bl0, bl1, bl2, bl3, bl4, bl5, bl6, bl7])}
  wreg8 = jnp.tile(Wreg, (1, 8))
  breg8 = jnp.broadcast_to(breg, (8,)).reshape(1, 8)

  def agg(d, table):
    if d == 128:
      pa = agg(64, table[:, :64])
      pb = agg(64, table[:, 64:])
      return jnp.concatenate([pa, pb], axis=2)
    ss, dd = idx[64 if d == 128 else 128]
    return _agg[d](table, ss, dd, zer[d])

  D = agg(16, ones16)
  P0 = agg(128, xp)

  h1a, h1b, dgv = _tc(
      _c0,
      [_node(128), _part(128), _part(16), _whole((128, 256)), _whole((1, 256)),
       _whole((128, 256))],
      [_node(128), _node(128), _node(16)],
      [jax.ShapeDtypeStruct((R, 128), f32)] * 2 +
      [jax.ShapeDtypeStruct((R, 16), f32)],
  )(xp, P0, D, Wl0, b[0], Wr0)

  def layer256(ha, hb, wl, bl_, wr, douts):
    pa = agg(128, ha)
    pb = agg(128, hb)
    return _tc(
        _c256,
        [_node(128), _node(128), _part(128), _part(128), _node(16),
         _whole((256, wl.shape[1])), _whole((1, wl.shape[1])),
         _whole((256, wl.shape[1]))],
        [_node(dd) for dd in douts],
        [jax.ShapeDtypeStruct((R, dd), f32) for dd in douts],
    )(ha, hb, pa, pb, dgv, wl, bl_, wr)

  def layern(hp, din, dout, wl, bl_, wr):
    pp = agg(din, hp)
    return _tc(
        _cn,
        [_node(din), _part(din), _node(16), _whole((din, dout)),
         _whole((1, dout)), _whole((din, dout))],
        _node(dout),
        jax.ShapeDtypeStruct((R, dout), f32),
    )(hp, pp, dgv, wl, bl_, wr)

  h2a, h2b = layer256(h1a, h1b, Wl1, b[1], Wr1, (128, 128))
  h3, = layer256(h2a, h2b, Wl2, b[2], Wr2, (128,))
  h4 = layern(h3, 128, 128, Wl3, b[3], Wr3)
  h5 = layern(h4, 128, 64, Wl4, b[4], Wr4)
  h6 = layern(h5, 64, 64, Wl5, b[5], Wr5)
  h7 = layern(h6, 64, 32, Wl6, b[6], Wr6)

  P7 = agg(32, h7)
  y8 = _tc(
      _c7,
      [_node(32), _part(32), _node(16), _whole((32, 32)), _whole((1, 32)),
       _whole((32, 32)), _whole((32, 8)), _whole((1, 8))],
      _node(8),
      jax.ShapeDtypeStruct((R, 8), f32),
  )(h7, P7, dgv, Wl7, b[7], Wr7, wreg8, breg8)

  return y8[:N, :1]
